# baseline (speedup 1.0000x reference)
import jax
import jax.numpy as jnp
from jax import lax
from jax.experimental import pallas as pl
from jax.experimental.pallas import tpu as pltpu
from jax.experimental.pallas import tpu_sc as plsc

NN = 10000
D = 256
E = 160000
TW = 160
NG = TW // 32
NC, NS = 2, 16
CH = 80
NCHUNK = 128
RING = 16
NRING = NCHUNK // RING
EPT = NCHUNK * CH
EPAD = EPT * NS
ACC_ROWS = 10112
STRIPE = ACC_ROWS // NS
TCR = 1000


def _tc1_body(n_ref, w1t_ref, w2_ref, t_ref):
    n = n_ref[...]
    h = jnp.tanh(jnp.dot(n, w1t_ref[...], preferred_element_type=jnp.float32))
    e = jnp.sum(h * w2_ref[...], axis=1, keepdims=True)
    q = jnp.exp(e)
    qn = q * n
    pad0 = jnp.zeros((TCR, TW - 128), jnp.float32)
    pad1 = jnp.zeros((TCR, TW - 129), jnp.float32)
    t_ref[0] = jnp.concatenate([qn[:, :128], pad0], axis=1).astype(jnp.bfloat16)
    t_ref[1] = jnp.concatenate([qn[:, 128:], q, pad1], axis=1).astype(jnp.bfloat16)


def _tc1(n, w1t, w2):
    return pl.pallas_call(
        _tc1_body,
        grid=(NN // TCR,),
        in_specs=[
            pl.BlockSpec((TCR, D), lambda i: (i, 0)),
            pl.BlockSpec((D, D), lambda i: (0, 0)),
            pl.BlockSpec((1, D), lambda i: (0, 0)),
        ],
        out_specs=pl.BlockSpec((2, TCR, TW), lambda i: (0, i, 0)),
        out_shape=jax.ShapeDtypeStruct((2, NN, TW), jnp.bfloat16),
    )(n, w1t, w2)


def _sc_body(t_hbm, src_hbm, dst_hbm, out_hbm, srcr, dstr, raw0, raw1, stage,
             acc, gsem0, gsem1):
    c = lax.axis_index("c")
    s = lax.axis_index("s")
    off = c * NN

    def _z(i, _):
        for j in range(TW // 16):
            stage[i, pl.ds(j * 16, 16)] = jnp.zeros((16,), jnp.float32)
        return 0
    lax.fori_loop(0, CH, _z, 0)
    base = s * STRIPE
    for r in range(0, STRIPE - CH + 1, CH):
        pltpu.sync_copy(stage, acc.at[pl.ds(base + r, CH)])
    rem = STRIPE % CH
    if rem:
        pltpu.sync_copy(stage.at[pl.ds(0, rem)],
                        acc.at[pl.ds(base + STRIPE - rem, rem)])
    plsc.subcore_barrier()

    raw = (raw0, raw1)
    gsem = (gsem0, gsem1)
    mask = jnp.full((16,), -65536, jnp.int32)

    def _make_widen(buf):
        def _widen(i4, _):
            for u in range(4):
                i = i4 * 4 + u
                for g in range(4):
                    x = buf[i, pl.ds(g * 16, 16)]
                    lo = plsc.bitcast(jnp.left_shift(x, 16), jnp.float32)
                    hi = plsc.bitcast(jnp.bitwise_and(x, mask), jnp.float32)
                    stage[i, pl.ds(g * 32, 16)] = lo
                    stage[i, pl.ds(g * 32 + 16, 16)] = hi
                x = buf[i, pl.ds(64, 16)]
                stage[i, pl.ds(128, 16)] = plsc.bitcast(
                    jnp.left_shift(x, 16), jnp.float32)
            return 0
        return _widen

    def _ring(r, _):
        ro = pl.multiple_of(r * RING, 8)
        pltpu.sync_copy(src_hbm.at[s].at[pl.ds(ro, RING)], srcr)
        pltpu.sync_copy(dst_hbm.at[s].at[pl.ds(ro, RING)], dstr)

        def _bias(i, _):
            for j in range(CH // 16):
                sl = pl.ds(j * 16, 16)
                srcr[i, sl] = srcr[i, sl] + off
            return 0
        lax.fori_loop(0, RING, _bias, 0)

        gd = [None] * RING
        gd[0] = pltpu.async_copy(t_hbm.at[srcr.at[0]], raw[0], gsem[0])
        for k in range(RING):
            b = k % 2
            if k + 1 < RING:
                gd[k + 1] = pltpu.async_copy(
                    t_hbm.at[srcr.at[k + 1]], raw[1 - b], gsem[1 - b])
            gd[k].wait()
            lax.fori_loop(0, CH // 4, _make_widen(raw[b]), 0)
            pltpu.sync_copy(stage, acc.at[dstr.at[k]], add=True)
        return 0
    lax.fori_loop(0, NRING, _ring, 0)
    plsc.subcore_barrier()

    pltpu.sync_copy(acc.at[pl.ds(base, STRIPE)],
                    out_hbm.at[c].at[pl.ds(base, STRIPE)])


def _sc_call():
    return pl.kernel(
        _sc_body,
        out_type=jax.ShapeDtypeStruct((NC, ACC_ROWS, TW), jnp.float32),
        mesh=plsc.VectorSubcoreMesh(
            core_axis_name="c", subcore_axis_name="s", num_cores=NC,
            num_subcores=NS),
        scratch_types=[
            pltpu.VMEM((RING, CH), jnp.int32),
            pltpu.VMEM((RING, CH), jnp.int32),
            pltpu.VMEM((CH, TW // 2), jnp.int32),
            pltpu.VMEM((CH, TW // 2), jnp.int32),
            pltpu.VMEM((CH, TW), jnp.float32),
            pltpu.VMEM_SHARED((ACC_ROWS, TW), jnp.float32),
            pltpu.SemaphoreType.DMA,
            pltpu.SemaphoreType.DMA,
        ],
        compiler_params=pltpu.CompilerParams(
            use_tc_tiling_on_sc=False, needs_layout_passes=False),
    )


def _tc2_body(o_ref, out_ref):
    o0 = o_ref[0]
    o1 = o_ref[1]
    ssum = o1[:, 128:129]
    inv = jnp.where(ssum > 0, 1.0 / ssum, 0.0)

    def _nat(oh):
        parts = []
        for g in range(4):
            lo = oh[:, 32 * g:32 * g + 16]
            hi = oh[:, 32 * g + 16:32 * g + 32]
            parts.append(
                jnp.stack([lo, hi], axis=2).reshape(lo.shape[0], 32))
        return jnp.concatenate(parts, axis=1)

    out_ref[...] = jnp.concatenate([_nat(o0) * inv, _nat(o1) * inv], axis=1)


def _tc2(o):
    return pl.pallas_call(
        _tc2_body,
        grid=(NN // TCR,),
        in_specs=[pl.BlockSpec((NC, TCR, TW), lambda i: (0, i, 0))],
        out_specs=pl.BlockSpec((TCR, D), lambda i: (i, 0)),
        out_shape=jax.ShapeDtypeStruct((NN, D), jnp.float32),
    )(o)


def kernel(N, edge_index, W1, W2):
    src = edge_index[0]
    dst = edge_index[1]
    pad = EPAD - E
    src3 = jnp.concatenate(
        [src, jnp.zeros((pad,), jnp.int32)]).reshape(NS, NCHUNK, CH)
    dst3 = jnp.concatenate(
        [dst, jnp.full((pad,), NN, jnp.int32)]).reshape(NS, NCHUNK, CH)
    t = _tc1(N, W1.T, W2)
    ti = lax.bitcast_convert_type(
        t.reshape(NC * NN, TW // 2, 2), jnp.int32)
    out = _sc_call()(ti, src3, dst3)
    return _tc2(out)


# speedup vs baseline: 2.9419x; 2.9419x over previous
import jax
import jax.numpy as jnp
from jax import lax
from jax.experimental import pallas as pl
from jax.experimental.pallas import tpu as pltpu
from jax.experimental.pallas import tpu_sc as plsc

NN = 10000
D = 256
E = 160000
HALF = 144
QCOL = 112
NC, NS = 2, 16
CH = 128
NCHUNK = 80
RING = 8
NRING = NCHUNK // RING
EPT = NCHUNK * CH
EPAD = EPT * NS
ACC_ROWS = 10112
STRIPE = ACC_ROWS // NS
TCR = 1000


def _tc1_body(n_ref, w1t_ref, w2_ref, t_ref):
    n = n_ref[...]
    h = jnp.tanh(jnp.dot(n, w1t_ref[...], preferred_element_type=jnp.float32))
    e = jnp.sum(h * w2_ref[...], axis=1, keepdims=True)
    q = jnp.exp(e)
    qn = q * n
    t_ref[0] = qn[:, :HALF]
    t_ref[1] = jnp.concatenate(
        [qn[:, HALF:], q, jnp.zeros((TCR, HALF - QCOL - 1), jnp.float32)],
        axis=1)


def _tc1(n, w1t, w2):
    return pl.pallas_call(
        _tc1_body,
        grid=(NN // TCR,),
        in_specs=[
            pl.BlockSpec((TCR, D), lambda i: (i, 0)),
            pl.BlockSpec((D, D), lambda i: (0, 0)),
            pl.BlockSpec((1, D), lambda i: (0, 0)),
        ],
        out_specs=pl.BlockSpec((2, TCR, HALF), lambda i: (0, i, 0)),
        out_shape=jax.ShapeDtypeStruct((2, NN, HALF), jnp.float32),
    )(n, w1t, w2)


def _sc_body(t_hbm, src_hbm, dst_hbm, out_hbm, srcr, dstr, rows0, rows1,
             acc, gsem0, gsem1):
    c = lax.axis_index("c")
    s = lax.axis_index("s")
    off = c * NN

    def _z(i, _):
        for j in range(HALF // 16):
            rows0[i, pl.ds(j * 16, 16)] = jnp.zeros((16,), jnp.float32)
        return 0
    lax.fori_loop(0, CH, _z, 0)
    base = s * STRIPE
    for r in range(0, STRIPE - CH + 1, CH):
        pltpu.sync_copy(rows0, acc.at[pl.ds(base + r, CH)])
    rem = STRIPE % CH
    if rem:
        pltpu.sync_copy(rows0.at[pl.ds(0, rem)],
                        acc.at[pl.ds(base + STRIPE - rem, rem)])
    plsc.subcore_barrier()

    def _ring(r, _):
        ro = pl.multiple_of(r * RING, 8)
        pltpu.sync_copy(src_hbm.at[s].at[pl.ds(ro, RING)], srcr)
        pltpu.sync_copy(dst_hbm.at[s].at[pl.ds(ro, RING)], dstr)

        def _bias(i, _):
            for j in range(CH // 16):
                sl = pl.ds(j * 16, 16)
                srcr[i, sl] = srcr[i, sl] + off
            return 0
        lax.fori_loop(0, RING, _bias, 0)

        rows = (rows0, rows1)
        gsem = (gsem0, gsem1)
        gd = [None] * RING
        gd[0] = pltpu.async_copy(t_hbm.at[srcr.at[0]], rows[0], gsem[0])
        for k in range(RING):
            b = k % 2
            if k + 1 < RING:
                gd[k + 1] = pltpu.async_copy(
                    t_hbm.at[srcr.at[k + 1]], rows[1 - b], gsem[1 - b])
            gd[k].wait()
            pltpu.sync_copy(rows[b], acc.at[dstr.at[k]], add=True)
        return 0
    lax.fori_loop(0, NRING, _ring, 0)
    plsc.subcore_barrier()

    pltpu.sync_copy(acc.at[pl.ds(base, STRIPE)],
                    out_hbm.at[c].at[pl.ds(base, STRIPE)])


def _sc_call():
    return pl.kernel(
        _sc_body,
        out_type=jax.ShapeDtypeStruct((NC, ACC_ROWS, HALF), jnp.float32),
        mesh=plsc.VectorSubcoreMesh(
            core_axis_name="c", subcore_axis_name="s", num_cores=NC,
            num_subcores=NS),
        scratch_types=[
            pltpu.VMEM((RING, CH), jnp.int32),
            pltpu.VMEM((RING, CH), jnp.int32),
            pltpu.VMEM((CH, HALF), jnp.float32),
            pltpu.VMEM((CH, HALF), jnp.float32),
            pltpu.VMEM_SHARED((ACC_ROWS, HALF), jnp.float32),
            pltpu.SemaphoreType.DMA,
            pltpu.SemaphoreType.DMA,
        ],
        compiler_params=pltpu.CompilerParams(use_tc_tiling_on_sc=False),
    )


def _tc2_body(o_ref, out_ref):
    o0 = o_ref[0]
    o1 = o_ref[1]
    ssum = o1[:, QCOL:QCOL + 1]
    inv = jnp.where(ssum > 0, 1.0 / ssum, 0.0)
    out_ref[...] = jnp.concatenate([o0 * inv, o1[:, :QCOL] * inv], axis=1)


def _tc2(o):
    return pl.pallas_call(
        _tc2_body,
        grid=(NN // TCR,),
        in_specs=[pl.BlockSpec((NC, TCR, HALF), lambda i: (0, i, 0))],
        out_specs=pl.BlockSpec((TCR, D), lambda i: (i, 0)),
        out_shape=jax.ShapeDtypeStruct((NN, D), jnp.float32),
    )(o)


def kernel(N, edge_index, W1, W2):
    src = edge_index[0]
    dst = edge_index[1]
    pad = EPAD - E
    src3 = jnp.concatenate(
        [src, jnp.zeros((pad,), jnp.int32)]).reshape(NS, NCHUNK, CH)
    dst3 = jnp.concatenate(
        [dst, jnp.full((pad,), NN, jnp.int32)]).reshape(NS, NCHUNK, CH)
    t = _tc1(N, W1.T, W2)
    out = _sc_call()(t.reshape(NC * NN, HALF), src3, dst3)
    return _tc2(out)
